# Initial kernel scaffold; baseline (speedup 1.0000x reference)
import jax
import jax.numpy as jnp
from jax.experimental import pallas as pl


def kernel(vaccine_embeddings, adjuvant_embeddings, positive_edges, negative_samples):
    raise NotImplementedError("write your pallas kernel here")


# speedup vs baseline: 3.6954x; 3.6954x over previous
import functools

import jax
import jax.numpy as jnp
from jax import lax
from jax.experimental import pallas as pl
from jax.experimental.pallas import tpu as pltpu
from jax.experimental.pallas import tpu_sc as plsc

E = 160000
D = 128
K = 4
NC, NS, L = 2, 16, 16
NW = NC * NS
C = 64
EPW = -(-E // (NW * C)) * C
EP = NW * EPW
NCH = EPW // C
G = C // L


def _sc_scores(vac, adj, pe0, pe1, negf):
    mesh = plsc.VectorSubcoreMesh(core_axis_name="c", subcore_axis_name="s")
    sds = jax.ShapeDtypeStruct((EP,), jnp.float32)

    @functools.partial(
        pl.kernel,
        mesh=mesh,
        out_type=[sds, sds, sds, sds, sds],
        compiler_params=pltpu.CompilerParams(needs_layout_passes=False),
        scratch_types=[
            pltpu.VMEM((C,), jnp.int32),
            pltpu.VMEM((C,), jnp.int32),
            pltpu.VMEM((2, 2 * C), jnp.int32),
            pltpu.VMEM((C, D), jnp.float32),
            pltpu.VMEM((C, D), jnp.float32),
            pltpu.VMEM((K * C, D), jnp.float32),
            pltpu.VMEM((EPW,), jnp.float32),
            pltpu.VMEM((EPW,), jnp.float32),
            pltpu.VMEM((EPW,), jnp.float32),
            pltpu.VMEM((EPW,), jnp.float32),
            pltpu.VMEM((EPW,), jnp.float32),
            pltpu.SemaphoreType.DMA,
        ],
    )
    def k(vac_h, adj_h, pe0_h, pe1_h, negf_h,
          op_h, o0_h, o1_h, o2_h, o3_h,
          idx0, idx1, idxn, pvb, pab, nbb, sp, s0, s1, s2, s3, sem):
        wid = lax.axis_index("s") * NC + lax.axis_index("c")
        base = wid * EPW

        def chunk_body(c, carry_none):
            b = base + c * C
            co = c * C
            pltpu.sync_copy(pe0_h.at[pl.ds(b, C)], idx0)
            pltpu.sync_copy(pe1_h.at[pl.ds(b, C)], idx1)
            pltpu.sync_copy(negf_h.at[pl.ds(K * b, 2 * C)], idxn.at[0])
            pltpu.sync_copy(negf_h.at[pl.ds(K * b + 2 * C, 2 * C)], idxn.at[1])
            h1 = pltpu.async_copy(vac_h.at[idx0], pvb, sem)
            h2 = pltpu.async_copy(adj_h.at[idx1], pab, sem)
            h3 = pltpu.async_copy(adj_h.at[idxn.at[0]], nbb.at[pl.ds(0, 2 * C)], sem)
            h4 = pltpu.async_copy(adj_h.at[idxn.at[1]], nbb.at[pl.ds(2 * C, 2 * C)], sem)
            h1.wait()
            h2.wait()
            h3.wait()
            h4.wait()

            mask_l0 = lax.iota(jnp.int32, L) == 0
            zf = jnp.zeros((L,), jnp.float32)

            def edge_body(e, inner_none):
                accp = zf
                a0 = zf
                a1 = zf
                a2 = zf
                a3 = zf
                r = e * K
                for j in range(D // L):
                    sl = pl.ds(j * L, L)
                    pv = pvb[e, sl]
                    accp = accp + pv * pab[e, sl]
                    a0 = a0 + pv * nbb[r + 0, sl]
                    a1 = a1 + pv * nbb[r + 1, sl]
                    a2 = a2 + pv * nbb[r + 2, sl]
                    a3 = a3 + pv * nbb[r + 3, sl]
                iv = jnp.full((L,), co + e, jnp.int32)
                for buf, acc in ((sp, accp), (s0, a0), (s1, a1),
                                 (s2, a2), (s3, a3)):
                    tot = lax.rev(jnp.cumsum(acc), (0,))
                    plsc.store_scatter(buf, [iv], tot, mask=mask_l0)
                return inner_none

            lax.fori_loop(0, C, edge_body, None, unroll=2)
            return carry_none

        lax.fori_loop(0, NCH, chunk_body, None)
        sl = pl.ds(base, EPW)
        pltpu.sync_copy(sp, op_h.at[sl])
        pltpu.sync_copy(s0, o0_h.at[sl])
        pltpu.sync_copy(s1, o1_h.at[sl])
        pltpu.sync_copy(s2, o2_h.at[sl])
        pltpu.sync_copy(s3, o3_h.at[sl])

    return k(vac, adj, pe0, pe1, negf)


def _epilogue(p, n0, n1, n2, n3):
    def body(p_ref, a0_ref, a1_ref, a2_ref, a3_ref, o_ref):
        ps = p_ref[...]
        a0 = a0_ref[...]
        a1 = a1_ref[...]
        a2 = a2_ref[...]
        a3 = a3_ref[...]
        pos = jnp.logaddexp(0.0, -ps)
        m = jnp.maximum(jnp.maximum(a0, a1), jnp.maximum(a2, a3))
        e0 = jnp.exp(a0 - m)
        e1 = jnp.exp(a1 - m)
        e2 = jnp.exp(a2 - m)
        e3 = jnp.exp(a3 - m)
        z = e0 + e1 + e2 + e3
        neg = (e0 * jnp.logaddexp(0.0, a0) + e1 * jnp.logaddexp(0.0, a1)
               + e2 * jnp.logaddexp(0.0, a2) + e3 * jnp.logaddexp(0.0, a3)) / z
        o_ref[...] = ((jnp.sum(pos) + jnp.sum(neg)) / E).reshape(1, 1)

    out = pl.pallas_call(
        body,
        out_shape=jax.ShapeDtypeStruct((1, 1), jnp.float32),
    )(p, n0, n1, n2, n3)
    return out[0, 0]


def kernel(vaccine_embeddings, adjuvant_embeddings, positive_edges, negative_samples):
    pad = EP - E
    pe0 = jnp.concatenate([positive_edges[:, 0], jnp.zeros((pad,), jnp.int32)])
    pe1 = jnp.concatenate([positive_edges[:, 1], jnp.zeros((pad,), jnp.int32)])
    negf = jnp.concatenate(
        [negative_samples.reshape(-1), jnp.zeros((K * pad,), jnp.int32)])
    sp, s0, s1, s2, s3 = _sc_scores(
        vaccine_embeddings, adjuvant_embeddings, pe0, pe1, negf)
    r = (E // D, D)
    return _epilogue(
        sp[:E].reshape(r), s0[:E].reshape(r), s1[:E].reshape(r),
        s2[:E].reshape(r), s3[:E].reshape(r))
